# Initial kernel scaffold; baseline (speedup 1.0000x reference)
import jax
import jax.numpy as jnp
from jax.experimental import pallas as pl


def kernel(pe, pos):
    raise NotImplementedError("write your pallas kernel here")


# speedup vs baseline: 1.9327x; 1.9327x over previous
import functools

import jax
import jax.numpy as jnp
from jax import lax
from jax.experimental import pallas as pl
from jax.experimental.pallas import tpu as pltpu
from jax.experimental.pallas import tpu_sc as plsc

MAX_LEN = 8192
DIM = 1024
N_POS = 16384

_info = plsc.get_sparse_core_info()
NC = _info.num_cores
NS = _info.num_subcores
NW = NC * NS
BPW = N_POS // NW
CHUNK = 64
NCHUNK = BPW // CHUNK

_mesh = plsc.VectorSubcoreMesh(core_axis_name="c", subcore_axis_name="s")


@functools.partial(
    pl.kernel,
    mesh=_mesh,
    out_type=jax.ShapeDtypeStruct((N_POS, DIM), jnp.float32),
    scratch_types=[
        pltpu.VMEM((NCHUNK, CHUNK), jnp.int32),
        pltpu.VMEM((CHUNK, DIM), jnp.float32),
        pltpu.SemaphoreType.DMA,
    ],
)
def _gather_kernel(pe_hbm, pos_hbm, out_hbm, idx_v, rows_v, sem):
    wid = lax.axis_index("s") * NC + lax.axis_index("c")
    pltpu.sync_copy(pos_hbm.at[wid], idx_v)
    base = wid * BPW
    for j in range(NCHUNK):
        pltpu.async_copy(pe_hbm.at[idx_v.at[j]], rows_v, sem).wait()
        pltpu.sync_copy(rows_v, out_hbm.at[pl.ds(base + j * CHUNK, CHUNK)])


def kernel(pe, pos):
    table = jnp.reshape(pe, (MAX_LEN, DIM))
    pos_r = jnp.reshape(pos, (NW, NCHUNK, CHUNK))
    return _gather_kernel(table, pos_r)


# speedup vs baseline: 2.0496x; 1.0605x over previous
import functools

import jax
import jax.numpy as jnp
from jax import lax
from jax.experimental import pallas as pl
from jax.experimental.pallas import tpu as pltpu
from jax.experimental.pallas import tpu_sc as plsc

MAX_LEN = 8192
DIM = 1024
N_POS = 16384

_info = plsc.get_sparse_core_info()
NC = _info.num_cores
NS = _info.num_subcores
NW = NC * NS
BPW = N_POS // NW
CHUNK = 32
NCHUNK = BPW // CHUNK
NBUF = 3

_mesh = plsc.VectorSubcoreMesh(core_axis_name="c", subcore_axis_name="s")


@functools.partial(
    pl.kernel,
    mesh=_mesh,
    out_type=jax.ShapeDtypeStruct((N_POS, DIM), jnp.float32),
    scratch_types=[
        pltpu.VMEM((NCHUNK, CHUNK), jnp.int32),
        *[pltpu.VMEM((CHUNK, DIM), jnp.float32) for _ in range(NBUF)],
        *[pltpu.SemaphoreType.DMA for _ in range(2 * NBUF)],
    ],
)
def _gather_kernel(pe_hbm, pos_hbm, out_hbm, idx_v, *bufs_and_sems):
    bufs = bufs_and_sems[:NBUF]
    gsem = bufs_and_sems[NBUF:2 * NBUF]
    wsem = bufs_and_sems[2 * NBUF:]
    wid = lax.axis_index("s") * NC + lax.axis_index("c")
    pltpu.sync_copy(pos_hbm.at[wid], idx_v)
    base = wid * BPW

    def gather(j, b):
        return pltpu.async_copy(pe_hbm.at[idx_v.at[j]], bufs[b], gsem[b])

    def write(j, b):
        return pltpu.async_copy(
            bufs[b], out_hbm.at[pl.ds(base + j * CHUNK, CHUNK)], wsem[b])

    g = [None] * NBUF
    w = [None] * NBUF
    for j in range(NBUF - 1):
        g[j] = gather(j, j)
    for j in range(NCHUNK):
        b = j % NBUF
        jn = j + NBUF - 1
        if jn < NCHUNK:
            bn = jn % NBUF
            if w[bn] is not None:
                w[bn].wait()
            g[bn] = gather(jn, bn)
        g[b].wait()
        w[b] = write(j, b)
    for b in range(NBUF):
        if w[b] is not None:
            w[b].wait()


def kernel(pe, pos):
    table = jnp.reshape(pe, (MAX_LEN, DIM))
    pos_r = jnp.reshape(pos, (NW, NCHUNK, CHUNK))
    return _gather_kernel(table, pos_r)


# speedup vs baseline: 2.0608x; 1.0055x over previous
import functools

import jax
import jax.numpy as jnp
from jax import lax
from jax.experimental import pallas as pl
from jax.experimental.pallas import tpu as pltpu
from jax.experimental.pallas import tpu_sc as plsc

MAX_LEN = 8192
DIM = 1024
N_POS = 16384

_info = plsc.get_sparse_core_info()
NC = _info.num_cores
NS = _info.num_subcores
NW = NC * NS
BPW = N_POS // NW
CHUNK = 16
NCHUNK = BPW // CHUNK
NBUF = 6

_mesh = plsc.VectorSubcoreMesh(core_axis_name="c", subcore_axis_name="s")


@functools.partial(
    pl.kernel,
    mesh=_mesh,
    out_type=jax.ShapeDtypeStruct((N_POS, DIM), jnp.float32),
    scratch_types=[
        pltpu.VMEM((NCHUNK, CHUNK), jnp.int32),
        *[pltpu.VMEM((CHUNK, DIM), jnp.float32) for _ in range(NBUF)],
        *[pltpu.SemaphoreType.DMA for _ in range(2 * NBUF)],
    ],
)
def _gather_kernel(pe_hbm, pos_hbm, out_hbm, idx_v, *bufs_and_sems):
    bufs = bufs_and_sems[:NBUF]
    gsem = bufs_and_sems[NBUF:2 * NBUF]
    wsem = bufs_and_sems[2 * NBUF:]
    wid = lax.axis_index("s") * NC + lax.axis_index("c")
    pltpu.sync_copy(pos_hbm.at[wid], idx_v)
    base = wid * BPW

    def gather(j, b):
        return pltpu.async_copy(pe_hbm.at[idx_v.at[j]], bufs[b], gsem[b])

    def write(j, b):
        return pltpu.async_copy(
            bufs[b], out_hbm.at[pl.ds(base + j * CHUNK, CHUNK)], wsem[b])

    g = [None] * NBUF
    w = [None] * NBUF
    for j in range(NBUF - 1):
        g[j] = gather(j, j)
    for j in range(NCHUNK):
        b = j % NBUF
        jn = j + NBUF - 1
        if jn < NCHUNK:
            bn = jn % NBUF
            if w[bn] is not None:
                w[bn].wait()
            g[bn] = gather(jn, bn)
        g[b].wait()
        w[b] = write(j, b)
    for b in range(NBUF):
        if w[b] is not None:
            w[b].wait()


def kernel(pe, pos):
    table = jnp.reshape(pe, (MAX_LEN, DIM))
    pos_r = jnp.reshape(pos, (NW, NCHUNK, CHUNK))
    return _gather_kernel(table, pos_r)
